# baseline (speedup 1.0000x reference)
import functools

import jax
import jax.numpy as jnp
from jax import lax
from jax.experimental import pallas as pl
from jax.experimental.pallas import tpu as pltpu
from jax.experimental.pallas import tpu_sc as plsc

NC = 2
NS = 16
NW = NC * NS
CH = 128


def _cdiv(a, b):
    return (a + b - 1) // b


NB = 4
NB1 = 8


def _make_scatter(n_pad, c1, dh):
    mesh = plsc.VectorSubcoreMesh(core_axis_name="c", subcore_axis_name="s")

    @functools.partial(
        pl.kernel,
        out_type=jax.ShapeDtypeStruct((NC * n_pad, dh), jnp.float32),
        mesh=mesh,
        scratch_types=[
            [pltpu.VMEM((2, CH), jnp.int32) for _ in range(NB1)],
            [pltpu.VMEM((CH, dh), jnp.float32) for _ in range(NB1)],
            pltpu.VMEM_SHARED((n_pad, dh), jnp.float32),
            [pltpu.SemaphoreType.DMA for _ in range(NB1)],
            [pltpu.SemaphoreType.DMA for _ in range(NB1)],
            [pltpu.SemaphoreType.DMA for _ in range(NB1)],
        ],
        compiler_params=pltpu.CompilerParams(use_tc_tiling_on_sc=False),
    )
    def scatter_k(x_hbm, idx_hbm, zeros_hbm, out_hbm,
                  idx_v, rows_v, acc_sh, isem, gsem, asem):
        cid = lax.axis_index("c")
        sid = lax.axis_index("s")
        wid = cid * NS + sid
        rpt = n_pad // NS
        row0 = pl.multiple_of(sid * rpt, 8)
        pltpu.sync_copy(zeros_hbm.at[pl.ds(row0, rpt)],
                        acc_sh.at[pl.ds(row0, rpt)])
        plsc.subcore_barrier()

        def body(g, carry):
            c0 = g * NB1
            di = [None] * NB1
            dg = [None] * NB1
            da = [None] * NB1
            for b in range(NB1):
                di[b] = pltpu.async_copy(idx_hbm.at[wid, c0 + b],
                                         idx_v[b], isem[b])
            for b in range(NB1):
                di[b].wait()
                dg[b] = pltpu.async_copy(x_hbm.at[idx_v[b].at[0]],
                                         rows_v[b], gsem[b])
            for b in range(NB1):
                dg[b].wait()
                da[b] = pltpu.async_copy(rows_v[b], acc_sh.at[idx_v[b].at[1]],
                                         asem[b], add=True)
            for b in range(NB1):
                da[b].wait()
            return carry

        lax.fori_loop(0, c1 // NB1, body, 0)
        plsc.subcore_barrier()
        pltpu.sync_copy(acc_sh.at[pl.ds(row0, rpt)],
                        out_hbm.at[pl.ds(pl.multiple_of(cid * n_pad + sid * rpt, 8),
                                         rpt)])

    return scatter_k


def _make_gather(n_pad, c2, d, e_pad):
    mesh = plsc.VectorSubcoreMesh(core_axis_name="c", subcore_axis_name="s")

    @functools.partial(
        pl.kernel,
        out_type=jax.ShapeDtypeStruct((e_pad, d), jnp.float32),
        mesh=mesh,
        scratch_types=[
            pltpu.VMEM((c2, CH), jnp.int32),
            pltpu.VMEM((c2, CH), jnp.int32),
            [pltpu.VMEM((CH, d), jnp.float32) for _ in range(NB)],
            [pltpu.SemaphoreType.DMA for _ in range(NB)],
            [pltpu.SemaphoreType.DMA for _ in range(NB)],
            [pltpu.SemaphoreType.DMA for _ in range(NB)],
        ],
    )
    def gather_k(y1_hbm, y2_hbm, sidx_hbm, ridx_hbm, out_hbm,
                 sidx_v, ridx_v, buf_v, g1sem, g2sem, stsem):
        cid = lax.axis_index("c")
        sid = lax.axis_index("s")
        wid = cid * NS + sid
        pltpu.sync_copy(sidx_hbm.at[wid], sidx_v)
        pltpu.sync_copy(ridx_hbm.at[wid], ridx_v)
        base = wid * c2 * CH

        def body(g, carry):
            c0 = g * NB
            d1 = [None] * NB
            d2 = [None] * NB
            d3 = [None] * NB
            for b in range(NB):
                d1[b] = pltpu.async_copy(y1_hbm.at[sidx_v.at[c0 + b]],
                                         buf_v[b], g1sem[b])
            for b in range(NB):
                d1[b].wait()
                d2[b] = pltpu.async_copy(y2_hbm.at[ridx_v.at[c0 + b]],
                                         buf_v[b], g2sem[b], add=True)
            for b in range(NB):
                d2[b].wait()
                d3[b] = pltpu.async_copy(
                    buf_v[b], out_hbm.at[pl.ds(base + (c0 + b) * CH, CH)],
                    stsem[b])
            for b in range(NB):
                d3[b].wait()
            return carry

        lax.fori_loop(0, c2 // NB, body, 0)

    return gather_k


def _node_matmul(agg, w1a, w1b, w2a, w2b, n_pad, dh, d):
    br = n_pad // 8
    grid = (8,)

    def body(pa_ref, pb_ref, w1a_ref, w1b_ref, w2a_ref, w2b_ref,
             y1_ref, y2_ref):
        pa = pa_ref[...]
        pb = pb_ref[...]
        y1_ref[...] = (jnp.dot(pa, w1a_ref[...], preferred_element_type=jnp.float32)
                       + jnp.dot(pb, w1b_ref[...], preferred_element_type=jnp.float32))
        y2_ref[...] = (jnp.dot(pa, w2a_ref[...], preferred_element_type=jnp.float32)
                       + jnp.dot(pb, w2b_ref[...], preferred_element_type=jnp.float32))

    wspec = pl.BlockSpec((dh, d), lambda i: (0, 0))
    y1, y2 = pl.pallas_call(
        body,
        grid=grid,
        in_specs=[
            pl.BlockSpec((br, dh), lambda i: (i, 0)),
            pl.BlockSpec((br, dh), lambda i: (i + 8, 0)),
            wspec, wspec, wspec, wspec,
        ],
        out_specs=[
            pl.BlockSpec((br, d), lambda i: (i, 0)),
            pl.BlockSpec((br, d), lambda i: (i, 0)),
        ],
        out_shape=[
            jax.ShapeDtypeStruct((n_pad, d), jnp.float32),
            jax.ShapeDtypeStruct((n_pad, d), jnp.float32),
        ],
    )(agg, agg, w1a, w1b, w2a, w2b)
    return y1, y2


def _edge_final(gpad, edge_attr, w3, b, n_edges, d_edge, d):
    br = 1280
    grid = (n_edges // br,)

    def body(g_ref, ea_ref, w3_ref, b_ref, o_ref):
        o_ref[...] = (g_ref[...]
                      + jnp.dot(ea_ref[...], w3_ref[...],
                                preferred_element_type=jnp.float32)
                      + b_ref[...])

    return pl.pallas_call(
        body,
        grid=grid,
        in_specs=[
            pl.BlockSpec((br, d), lambda i: (i, 0)),
            pl.BlockSpec((br, d_edge), lambda i: (i, 0)),
            pl.BlockSpec((d_edge, d), lambda i: (0, 0)),
            pl.BlockSpec((1, d), lambda i: (0, 0)),
        ],
        out_specs=pl.BlockSpec((br, d), lambda i: (i, 0)),
        out_shape=jax.ShapeDtypeStruct((n_edges, d), jnp.float32),
    )(gpad, edge_attr, w3, b)


def kernel(x, edge_index, edge_attr, W, b):
    n_nodes, d = x.shape
    n_edges, d_edge = edge_attr.shape

    dh = d // 2
    s = edge_index[0].astype(jnp.int32)
    r = edge_index[1].astype(jnp.int32)

    m = 2 * n_edges
    c1 = NB1 * _cdiv(m, NS * CH * NB1)
    m_pad = NS * c1 * CH
    gidx = jnp.concatenate([r, s])
    sidx = jnp.concatenate([s, r])
    gidx = jnp.concatenate([gidx, jnp.zeros((m_pad - m,), jnp.int32)])
    sidx = jnp.concatenate([sidx, jnp.full((m_pad - m,), n_nodes, jnp.int32)])
    gidx3 = jnp.concatenate([gidx, gidx + n_nodes]).reshape(NW, c1, CH)
    sidx3 = jnp.concatenate([sidx, sidx]).reshape(NW, c1, CH)
    idxcat = jnp.stack([gidx3, sidx3], axis=2)
    xcat = jnp.concatenate([x[:, :dh], x[:, dh:]], axis=0)

    n_pad = 128 * _cdiv(n_nodes + 1, 128)
    zeros = jnp.zeros((n_pad, dh), jnp.float32)

    agg = _make_scatter(n_pad, c1, dh)(xcat, idxcat, zeros)

    y1, y2 = _node_matmul(agg, W[:dh], W[dh:d], W[d:d + dh], W[d + dh:2 * d],
                          n_pad, dh, d)

    c2 = NB * _cdiv(n_edges, NW * CH * NB)
    e_pad = NW * c2 * CH
    sp = jnp.concatenate([s, jnp.zeros((e_pad - n_edges,), jnp.int32)])
    rp = jnp.concatenate([r, jnp.zeros((e_pad - n_edges,), jnp.int32)])
    gpad = _make_gather(n_pad, c2, d, e_pad)(
        y1, y2, sp.reshape(NW, c2, CH), rp.reshape(NW, c2, CH))

    return _edge_final(gpad, edge_attr, W[2 * d:], b.reshape(1, d),
                       n_edges, d_edge, d)


# speedup vs baseline: 1.0946x; 1.0946x over previous
import functools

import jax
import jax.numpy as jnp
from jax import lax
from jax.experimental import pallas as pl
from jax.experimental.pallas import tpu as pltpu
from jax.experimental.pallas import tpu_sc as plsc

NC = 2
NS = 16
NW = NC * NS
CH = 128


def _cdiv(a, b):
    return (a + b - 1) // b


NB = 4
NB1 = 8


def _make_scatter(n_pad, c1, dh):
    mesh = plsc.VectorSubcoreMesh(core_axis_name="c", subcore_axis_name="s")

    @functools.partial(
        pl.kernel,
        out_type=jax.ShapeDtypeStruct((NC * n_pad, dh), jnp.float32),
        mesh=mesh,
        scratch_types=[
            pltpu.VMEM((c1, 2, CH), jnp.int32),
            pltpu.VMEM((CH, dh), jnp.float32),
            pltpu.VMEM_SHARED((n_pad, dh), jnp.float32),
            pltpu.SemaphoreType.DMA,
            pltpu.SemaphoreType.DMA,
        ],
        compiler_params=pltpu.CompilerParams(use_tc_tiling_on_sc=False),
    )
    def scatter_k(x_hbm, idx_hbm, zeros_hbm, out_hbm,
                  idx_v, rows_v, acc_sh, isem, gsem):
        cid = lax.axis_index("c")
        sid = lax.axis_index("s")
        wid = cid * NS + sid
        rpt = n_pad // NS
        row0 = pl.multiple_of(sid * rpt, 8)
        pltpu.sync_copy(zeros_hbm.at[pl.ds(row0, rpt)],
                        acc_sh.at[pl.ds(row0, rpt)])
        pltpu.sync_copy(idx_hbm.at[wid], idx_v)
        plsc.subcore_barrier()

        def body(ch, carry):
            pltpu.async_copy(x_hbm.at[idx_v.at[ch, 0]], rows_v, gsem).wait()
            pltpu.sync_copy(rows_v, acc_sh.at[idx_v.at[ch, 1]], add=True)
            return carry

        lax.fori_loop(0, c1, body, 0)
        plsc.subcore_barrier()
        pltpu.sync_copy(acc_sh.at[pl.ds(row0, rpt)],
                        out_hbm.at[pl.ds(pl.multiple_of(cid * n_pad + sid * rpt, 8),
                                         rpt)])

    return scatter_k


def _make_gather(n_pad, c2, d, e_pad):
    mesh = plsc.VectorSubcoreMesh(core_axis_name="c", subcore_axis_name="s")

    @functools.partial(
        pl.kernel,
        out_type=jax.ShapeDtypeStruct((e_pad, d), jnp.float32),
        mesh=mesh,
        scratch_types=[
            pltpu.VMEM((c2, CH), jnp.int32),
            pltpu.VMEM((c2, CH), jnp.int32),
            [pltpu.VMEM((CH, d), jnp.float32) for _ in range(2)],
            pltpu.SemaphoreType.DMA,
            pltpu.SemaphoreType.DMA,
            [pltpu.SemaphoreType.DMA for _ in range(2)],
        ],
    )
    def gather_k(y1_hbm, y2_hbm, sidx_hbm, ridx_hbm, out_hbm,
                 sidx_v, ridx_v, buf_v, g1sem, g2sem, stsem):
        cid = lax.axis_index("c")
        sid = lax.axis_index("s")
        wid = cid * NS + sid
        pltpu.sync_copy(sidx_hbm.at[wid], sidx_v)
        pltpu.sync_copy(ridx_hbm.at[wid], ridx_v)
        base = wid * c2 * CH

        def body(g, carry):
            st = [None, None]
            for b in range(2):
                ch = g * 2 + b
                pltpu.async_copy(y1_hbm.at[sidx_v.at[ch]], buf_v[b],
                                 g1sem).wait()
                pltpu.async_copy(y2_hbm.at[ridx_v.at[ch]], buf_v[b],
                                 g2sem, add=True).wait()
                st[b] = pltpu.async_copy(
                    buf_v[b], out_hbm.at[pl.ds(base + ch * CH, CH)], stsem[b])
            st[0].wait()
            st[1].wait()
            return carry

        lax.fori_loop(0, c2 // 2, body, 0)

    return gather_k


def _node_matmul(agg, w1a, w1b, w2a, w2b, n_pad, dh, d):
    br = n_pad // 8
    grid = (8,)

    def body(pa_ref, pb_ref, w1a_ref, w1b_ref, w2a_ref, w2b_ref,
             y1_ref, y2_ref):
        pa = pa_ref[...]
        pb = pb_ref[...]
        y1_ref[...] = (jnp.dot(pa, w1a_ref[...], preferred_element_type=jnp.float32)
                       + jnp.dot(pb, w1b_ref[...], preferred_element_type=jnp.float32))
        y2_ref[...] = (jnp.dot(pa, w2a_ref[...], preferred_element_type=jnp.float32)
                       + jnp.dot(pb, w2b_ref[...], preferred_element_type=jnp.float32))

    wspec = pl.BlockSpec((dh, d), lambda i: (0, 0))
    y1, y2 = pl.pallas_call(
        body,
        grid=grid,
        in_specs=[
            pl.BlockSpec((br, dh), lambda i: (i, 0)),
            pl.BlockSpec((br, dh), lambda i: (i + 8, 0)),
            wspec, wspec, wspec, wspec,
        ],
        out_specs=[
            pl.BlockSpec((br, d), lambda i: (i, 0)),
            pl.BlockSpec((br, d), lambda i: (i, 0)),
        ],
        out_shape=[
            jax.ShapeDtypeStruct((n_pad, d), jnp.float32),
            jax.ShapeDtypeStruct((n_pad, d), jnp.float32),
        ],
    )(agg, agg, w1a, w1b, w2a, w2b)
    return y1, y2


def _edge_final(gpad, edge_attr, w3, b, n_edges, d_edge, d):
    br = 1280
    grid = (n_edges // br,)

    def body(g_ref, ea_ref, w3_ref, b_ref, o_ref):
        o_ref[...] = (g_ref[...]
                      + jnp.dot(ea_ref[...], w3_ref[...],
                                preferred_element_type=jnp.float32)
                      + b_ref[...])

    return pl.pallas_call(
        body,
        grid=grid,
        in_specs=[
            pl.BlockSpec((br, d), lambda i: (i, 0)),
            pl.BlockSpec((br, d_edge), lambda i: (i, 0)),
            pl.BlockSpec((d_edge, d), lambda i: (0, 0)),
            pl.BlockSpec((1, d), lambda i: (0, 0)),
        ],
        out_specs=pl.BlockSpec((br, d), lambda i: (i, 0)),
        out_shape=jax.ShapeDtypeStruct((n_edges, d), jnp.float32),
    )(gpad, edge_attr, w3, b)


def kernel(x, edge_index, edge_attr, W, b):
    n_nodes, d = x.shape
    n_edges, d_edge = edge_attr.shape

    dh = d // 2
    s = edge_index[0].astype(jnp.int32)
    r = edge_index[1].astype(jnp.int32)

    m = 2 * n_edges
    c1 = _cdiv(m, NS * CH)
    m_pad = NS * c1 * CH
    gidx = jnp.concatenate([r, s])
    sidx = jnp.concatenate([s, r])
    gidx = jnp.concatenate([gidx, jnp.zeros((m_pad - m,), jnp.int32)])
    sidx = jnp.concatenate([sidx, jnp.full((m_pad - m,), n_nodes, jnp.int32)])
    gidx3 = jnp.concatenate([gidx, gidx + n_nodes]).reshape(NW, c1, CH)
    sidx3 = jnp.concatenate([sidx, sidx]).reshape(NW, c1, CH)
    idxcat = jnp.stack([gidx3, sidx3], axis=2)
    xcat = jnp.concatenate([x[:, :dh], x[:, dh:]], axis=0)

    n_pad = 128 * _cdiv(n_nodes + 1, 128)
    zeros = jnp.zeros((n_pad, dh), jnp.float32)

    agg = _make_scatter(n_pad, c1, dh)(xcat, idxcat, zeros)

    y1, y2 = _node_matmul(agg, W[:dh], W[dh:d], W[d:d + dh], W[d + dh:2 * d],
                          n_pad, dh, d)

    c2 = 2 * _cdiv(n_edges, NW * CH * 2)
    e_pad = NW * c2 * CH
    sp = jnp.concatenate([s, jnp.zeros((e_pad - n_edges,), jnp.int32)])
    rp = jnp.concatenate([r, jnp.zeros((e_pad - n_edges,), jnp.int32)])
    gpad = _make_gather(n_pad, c2, d, e_pad)(
        y1, y2, sp.reshape(NW, c2, CH), rp.reshape(NW, c2, CH))

    return _edge_final(gpad, edge_attr, W[2 * d:], b.reshape(1, d),
                       n_edges, d_edge, d)


# speedup vs baseline: 1.1168x; 1.0203x over previous
import functools

import jax
import jax.numpy as jnp
from jax import lax
from jax.experimental import pallas as pl
from jax.experimental.pallas import tpu as pltpu
from jax.experimental.pallas import tpu_sc as plsc

NC = 2
NS = 16
NW = NC * NS
CH = 128


def _cdiv(a, b):
    return (a + b - 1) // b


NB = 4
NB1 = 8


def _make_scatter(n_pad, c1, dh):
    mesh = plsc.VectorSubcoreMesh(core_axis_name="c", subcore_axis_name="s")

    @functools.partial(
        pl.kernel,
        out_type=jax.ShapeDtypeStruct((NC * n_pad, dh), jnp.float32),
        mesh=mesh,
        scratch_types=[
            pltpu.VMEM((c1, 2, CH), jnp.int32),
            pltpu.VMEM((CH, dh), jnp.float32),
            pltpu.VMEM_SHARED((n_pad, dh), jnp.float32),
            pltpu.SemaphoreType.DMA,
            pltpu.SemaphoreType.DMA,
        ],
        compiler_params=pltpu.CompilerParams(use_tc_tiling_on_sc=False),
    )
    def scatter_k(x_hbm, idx_hbm, zeros_hbm, out_hbm,
                  idx_v, rows_v, acc_sh, isem, gsem):
        cid = lax.axis_index("c")
        sid = lax.axis_index("s")
        wid = cid * NS + sid
        rpt = n_pad // NS
        row0 = pl.multiple_of(sid * rpt, 8)
        pltpu.sync_copy(zeros_hbm.at[pl.ds(row0, rpt)],
                        acc_sh.at[pl.ds(row0, rpt)])
        pltpu.sync_copy(idx_hbm.at[wid], idx_v)
        plsc.subcore_barrier()

        def body(ch, carry):
            pltpu.async_copy(x_hbm.at[idx_v.at[ch, 0]], rows_v, gsem).wait()
            pltpu.sync_copy(rows_v, acc_sh.at[idx_v.at[ch, 1]], add=True)
            return carry

        lax.fori_loop(0, c1, body, 0)
        plsc.subcore_barrier()
        pltpu.sync_copy(acc_sh.at[pl.ds(row0, rpt)],
                        out_hbm.at[pl.ds(pl.multiple_of(cid * n_pad + sid * rpt, 8),
                                         rpt)])

    return scatter_k


CH2 = 512


def _make_gather(n_pad, c2, d, e_pad):
    mesh = plsc.VectorSubcoreMesh(core_axis_name="c", subcore_axis_name="s")

    @functools.partial(
        pl.kernel,
        out_type=jax.ShapeDtypeStruct((e_pad, d), jnp.float32),
        mesh=mesh,
        scratch_types=[
            pltpu.VMEM((c2, CH2), jnp.int32),
            pltpu.VMEM((c2, CH2), jnp.int32),
            pltpu.VMEM((CH2, d), jnp.float32),
            pltpu.SemaphoreType.DMA,
        ],
        compiler_params=pltpu.CompilerParams(use_tc_tiling_on_sc=False),
    )
    def gather_k(y1_hbm, y2_hbm, sidx_hbm, ridx_hbm, out_hbm,
                 sidx_v, ridx_v, buf_v, sem):
        cid = lax.axis_index("c")
        sid = lax.axis_index("s")
        wid = cid * NS + sid
        pltpu.sync_copy(sidx_hbm.at[wid], sidx_v)
        pltpu.sync_copy(ridx_hbm.at[wid], ridx_v)
        base = wid * c2 * CH2

        def body(ch, carry):
            pltpu.async_copy(y1_hbm.at[sidx_v.at[ch]], buf_v, sem).wait()
            pltpu.async_copy(y2_hbm.at[ridx_v.at[ch]], buf_v, sem,
                             add=True).wait()
            pltpu.sync_copy(buf_v, out_hbm.at[pl.ds(base + ch * CH2, CH2)])
            return carry

        lax.fori_loop(0, c2, body, 0)

    return gather_k


def _node_matmul(agg, w1a, w1b, w2a, w2b, n_pad, dh, d):
    br = n_pad // 8
    grid = (8,)

    def body(pa_ref, pb_ref, w1a_ref, w1b_ref, w2a_ref, w2b_ref,
             y1_ref, y2_ref):
        pa = pa_ref[...]
        pb = pb_ref[...]
        y1_ref[...] = (jnp.dot(pa, w1a_ref[...], preferred_element_type=jnp.float32)
                       + jnp.dot(pb, w1b_ref[...], preferred_element_type=jnp.float32))
        y2_ref[...] = (jnp.dot(pa, w2a_ref[...], preferred_element_type=jnp.float32)
                       + jnp.dot(pb, w2b_ref[...], preferred_element_type=jnp.float32))

    wspec = pl.BlockSpec((dh, d), lambda i: (0, 0))
    y1, y2 = pl.pallas_call(
        body,
        grid=grid,
        in_specs=[
            pl.BlockSpec((br, dh), lambda i: (i, 0)),
            pl.BlockSpec((br, dh), lambda i: (i + 8, 0)),
            wspec, wspec, wspec, wspec,
        ],
        out_specs=[
            pl.BlockSpec((br, d), lambda i: (i, 0)),
            pl.BlockSpec((br, d), lambda i: (i, 0)),
        ],
        out_shape=[
            jax.ShapeDtypeStruct((n_pad, d), jnp.float32),
            jax.ShapeDtypeStruct((n_pad, d), jnp.float32),
        ],
    )(agg, agg, w1a, w1b, w2a, w2b)
    return y1, y2


def _edge_final(gpad, edge_attr, w3, b, n_edges, d_edge, d):
    br = 1280
    grid = (n_edges // br,)

    def body(g_ref, ea_ref, w3_ref, b_ref, o_ref):
        o_ref[...] = (g_ref[...]
                      + jnp.dot(ea_ref[...], w3_ref[...],
                                preferred_element_type=jnp.float32)
                      + b_ref[...])

    return pl.pallas_call(
        body,
        grid=grid,
        in_specs=[
            pl.BlockSpec((br, d), lambda i: (i, 0)),
            pl.BlockSpec((br, d_edge), lambda i: (i, 0)),
            pl.BlockSpec((d_edge, d), lambda i: (0, 0)),
            pl.BlockSpec((1, d), lambda i: (0, 0)),
        ],
        out_specs=pl.BlockSpec((br, d), lambda i: (i, 0)),
        out_shape=jax.ShapeDtypeStruct((n_edges, d), jnp.float32),
    )(gpad, edge_attr, w3, b)


def kernel(x, edge_index, edge_attr, W, b):
    n_nodes, d = x.shape
    n_edges, d_edge = edge_attr.shape

    dh = d // 2
    s = edge_index[0].astype(jnp.int32)
    r = edge_index[1].astype(jnp.int32)

    m = 2 * n_edges
    c1 = _cdiv(m, NS * CH)
    m_pad = NS * c1 * CH
    gidx = jnp.concatenate([r, s])
    sidx = jnp.concatenate([s, r])
    gidx = jnp.concatenate([gidx, jnp.zeros((m_pad - m,), jnp.int32)])
    sidx = jnp.concatenate([sidx, jnp.full((m_pad - m,), n_nodes, jnp.int32)])
    gidx3 = jnp.concatenate([gidx, gidx + n_nodes]).reshape(NW, c1, CH)
    sidx3 = jnp.concatenate([sidx, sidx]).reshape(NW, c1, CH)
    idxcat = jnp.stack([gidx3, sidx3], axis=2)
    xcat = jnp.concatenate([x[:, :dh], x[:, dh:]], axis=0)

    n_pad = 128 * _cdiv(n_nodes + 1, 128)
    zeros = jnp.zeros((n_pad, dh), jnp.float32)

    agg = _make_scatter(n_pad, c1, dh)(xcat, idxcat, zeros)

    y1, y2 = _node_matmul(agg, W[:dh], W[dh:d], W[d:d + dh], W[d + dh:2 * d],
                          n_pad, dh, d)

    c2 = _cdiv(n_edges, NW * CH2)
    e_pad = NW * c2 * CH2
    sp = jnp.concatenate([s, jnp.zeros((e_pad - n_edges,), jnp.int32)])
    rp = jnp.concatenate([r, jnp.zeros((e_pad - n_edges,), jnp.int32)])
    gpad = _make_gather(n_pad, c2, d, e_pad)(
        y1, y2, sp.reshape(NW, c2, CH2), rp.reshape(NW, c2, CH2))

    return _edge_final(gpad, edge_attr, W[2 * d:], b.reshape(1, d),
                       n_edges, d_edge, d)


# speedup vs baseline: 1.4454x; 1.2942x over previous
import functools

import jax
import jax.numpy as jnp
from jax import lax
from jax.experimental import pallas as pl
from jax.experimental.pallas import tpu as pltpu
from jax.experimental.pallas import tpu_sc as plsc

NC = 2
NS = 16
NW = NC * NS
CH = 128


def _cdiv(a, b):
    return (a + b - 1) // b


NB = 4
NB1 = 8


def _make_scatter(n_pad, c1, dh):
    mesh = plsc.VectorSubcoreMesh(core_axis_name="c", subcore_axis_name="s")

    @functools.partial(
        pl.kernel,
        out_type=jax.ShapeDtypeStruct((NC * n_pad, dh), jnp.float32),
        mesh=mesh,
        scratch_types=[
            pltpu.VMEM((c1, 2, CH), jnp.int32),
            pltpu.VMEM((CH, dh), jnp.float32),
            pltpu.VMEM_SHARED((n_pad, dh), jnp.float32),
            pltpu.SemaphoreType.DMA,
            pltpu.SemaphoreType.DMA,
        ],
        compiler_params=pltpu.CompilerParams(use_tc_tiling_on_sc=False),
    )
    def scatter_k(x_hbm, idx_hbm, zeros_hbm, out_hbm,
                  idx_v, rows_v, acc_sh, isem, gsem):
        cid = lax.axis_index("c")
        sid = lax.axis_index("s")
        wid = cid * NS + sid
        rpt = n_pad // NS
        row0 = pl.multiple_of(sid * rpt, 8)
        pltpu.sync_copy(zeros_hbm.at[pl.ds(row0, rpt)],
                        acc_sh.at[pl.ds(row0, rpt)])
        pltpu.sync_copy(idx_hbm.at[wid], idx_v)
        plsc.subcore_barrier()

        def body(ch, carry):
            pltpu.async_copy(x_hbm.at[idx_v.at[ch, 0]], rows_v, gsem).wait()
            pltpu.sync_copy(rows_v, acc_sh.at[idx_v.at[ch, 1]], add=True)
            return carry

        lax.fori_loop(0, c1, body, 0)
        plsc.subcore_barrier()
        pltpu.sync_copy(acc_sh.at[pl.ds(row0, rpt)],
                        out_hbm.at[pl.ds(pl.multiple_of(cid * n_pad + sid * rpt, 8),
                                         rpt)])

    return scatter_k


CH2 = 512
C2A = 32
C2B = 8


def _make_gather(n_pad, d, e_pad):
    mesh = plsc.VectorSubcoreMesh(core_axis_name="c", subcore_axis_name="s")

    @functools.partial(
        pl.kernel,
        out_type=jax.ShapeDtypeStruct((e_pad, d), jnp.float32),
        mesh=mesh,
        scratch_types=[
            pltpu.VMEM((C2A, CH2), jnp.int32),
            pltpu.VMEM((C2A, CH2), jnp.int32),
            pltpu.VMEM((CH2, d), jnp.float32),
            pltpu.SemaphoreType.DMA,
        ],
        compiler_params=pltpu.CompilerParams(use_tc_tiling_on_sc=False),
    )
    def gather_k(y1_hbm, y2_hbm, sidx_hbm, ridx_hbm, out_hbm,
                 sidx_v, ridx_v, buf_v, sem):
        cid = lax.axis_index("c")
        sid = lax.axis_index("s")
        wid = cid * NS + sid
        pltpu.sync_copy(sidx_hbm.at[wid], sidx_v)
        pltpu.sync_copy(ridx_hbm.at[wid], ridx_v)
        nch = jnp.where(cid == 0, C2A, C2B)
        base = cid * NS * C2A * CH2 + sid * nch * CH2

        def body(ch, carry):
            pltpu.async_copy(y1_hbm.at[sidx_v.at[ch]], buf_v, sem).wait()
            pltpu.async_copy(y2_hbm.at[ridx_v.at[ch]], buf_v, sem,
                             add=True).wait()
            pltpu.sync_copy(buf_v, out_hbm.at[pl.ds(base + ch * CH2, CH2)])
            return carry

        lax.fori_loop(0, nch, body, 0)

    return gather_k


def _node_matmul(agg, w1a, w1b, w2a, w2b, n_pad, dh, d):
    br = n_pad // 8
    grid = (8,)

    def body(pa_ref, pb_ref, w1a_ref, w1b_ref, w2a_ref, w2b_ref,
             y1_ref, y2_ref):
        pa = pa_ref[...]
        pb = pb_ref[...]
        y1_ref[...] = (jnp.dot(pa, w1a_ref[...], preferred_element_type=jnp.float32)
                       + jnp.dot(pb, w1b_ref[...], preferred_element_type=jnp.float32))
        y2_ref[...] = (jnp.dot(pa, w2a_ref[...], preferred_element_type=jnp.float32)
                       + jnp.dot(pb, w2b_ref[...], preferred_element_type=jnp.float32))

    wspec = pl.BlockSpec((dh, d), lambda i: (0, 0))
    y1, y2 = pl.pallas_call(
        body,
        grid=grid,
        in_specs=[
            pl.BlockSpec((br, dh), lambda i: (i, 0)),
            pl.BlockSpec((br, dh), lambda i: (i + 8, 0)),
            wspec, wspec, wspec, wspec,
        ],
        out_specs=[
            pl.BlockSpec((br, d), lambda i: (i, 0)),
            pl.BlockSpec((br, d), lambda i: (i, 0)),
        ],
        out_shape=[
            jax.ShapeDtypeStruct((n_pad, d), jnp.float32),
            jax.ShapeDtypeStruct((n_pad, d), jnp.float32),
        ],
    )(agg, agg, w1a, w1b, w2a, w2b)
    return y1, y2


def _edge_final(gpad, edge_attr, w3, b, n_edges, d_edge, d):
    br = 1280
    grid = (n_edges // br,)

    def body(g_ref, ea_ref, w3_ref, b_ref, o_ref):
        o_ref[...] = (g_ref[...]
                      + jnp.dot(ea_ref[...], w3_ref[...],
                                preferred_element_type=jnp.float32)
                      + b_ref[...])

    return pl.pallas_call(
        body,
        grid=grid,
        in_specs=[
            pl.BlockSpec((br, d), lambda i: (i, 0)),
            pl.BlockSpec((br, d_edge), lambda i: (i, 0)),
            pl.BlockSpec((d_edge, d), lambda i: (0, 0)),
            pl.BlockSpec((1, d), lambda i: (0, 0)),
        ],
        out_specs=pl.BlockSpec((br, d), lambda i: (i, 0)),
        out_shape=jax.ShapeDtypeStruct((n_edges, d), jnp.float32),
    )(gpad, edge_attr, w3, b)


def kernel(x, edge_index, edge_attr, W, b):
    n_nodes, d = x.shape
    n_edges, d_edge = edge_attr.shape

    dh = d // 2
    s = edge_index[0].astype(jnp.int32)
    r = edge_index[1].astype(jnp.int32)

    m = 2 * n_edges
    c1 = _cdiv(m, NS * CH)
    m_pad = NS * c1 * CH
    gidx = jnp.concatenate([r, s])
    sidx = jnp.concatenate([s, r])
    gidx = jnp.concatenate([gidx, jnp.zeros((m_pad - m,), jnp.int32)])
    sidx = jnp.concatenate([sidx, jnp.full((m_pad - m,), n_nodes, jnp.int32)])
    gidx3 = jnp.concatenate([gidx, gidx + n_nodes]).reshape(NW, c1, CH)
    sidx3 = jnp.concatenate([sidx, sidx]).reshape(NW, c1, CH)
    idxcat = jnp.stack([gidx3, sidx3], axis=2)
    xcat = jnp.concatenate([x[:, :dh], x[:, dh:]], axis=0)

    n_pad = 128 * _cdiv(n_nodes + 1, 128)
    zeros = jnp.zeros((n_pad, dh), jnp.float32)

    agg = _make_scatter(n_pad, c1, dh)(xcat, idxcat, zeros)

    y1, y2 = _node_matmul(agg, W[:dh], W[dh:d], W[d:d + dh], W[d + dh:2 * d],
                          n_pad, dh, d)

    e_pad = NS * (C2A + C2B) * CH2
    ea = NS * C2A * CH2
    sp = jnp.concatenate([s, jnp.zeros((e_pad - n_edges,), jnp.int32)])
    rp = jnp.concatenate([r, jnp.zeros((e_pad - n_edges,), jnp.int32)])

    def _split(v):
        va = v[:ea].reshape(NS, C2A, CH2)
        vb = v[ea:].reshape(NS, C2B, CH2)
        vb = jnp.pad(vb, ((0, 0), (0, C2A - C2B), (0, 0)))
        return jnp.concatenate([va, vb], axis=0)

    gpad = _make_gather(n_pad, d, e_pad)(y1, y2, _split(sp), _split(rp))

    return _edge_final(gpad, edge_attr, W[2 * d:], b.reshape(1, d),
                       n_edges, d_edge, d)
